# baseline (speedup 1.0000x reference)
import functools
import math

import jax
import jax.numpy as jnp
import numpy as np
from jax import lax
from jax.experimental import pallas as pl
from jax.experimental.pallas import tpu as pltpu
from jax.experimental.pallas import tpu_sc as plsc

NUM_BUCKETS = 32
RP_MAX_DISTANCE = 128
HEADS = 16
RP_SCALE = 0.125
SEQ_LEN = 2048
EXT2 = 6400
BLOCK_R = 256
NC, NS = 2, 16
NW = NC * NS
QS = EXT2 // NW


def _bucket_table() -> np.ndarray:
    q = np.arange(EXT2, dtype=np.int64)
    n = np.maximum(SEQ_LEN - q, 0).astype(np.int64)
    max_exact = NUM_BUCKETS // 2
    nf = np.maximum(n, 1).astype(np.float64)
    val_if_large = max_exact + (
        np.log(nf / max_exact) / math.log(RP_MAX_DISTANCE / max_exact)
        * (NUM_BUCKETS - max_exact)
    ).astype(np.int64)
    val_if_large = np.minimum(val_if_large, NUM_BUCKETS - 1)
    return np.where(n < max_exact, n, val_if_large).astype(np.int32)


_BUCKET = _bucket_table()

_SC_MESH = plsc.VectorSubcoreMesh(core_axis_name="c", subcore_axis_name="s")


@functools.partial(
    pl.kernel,
    mesh=_SC_MESH,
    out_type=jax.ShapeDtypeStruct((EXT2, HEADS), jnp.float32),
    compiler_params=pltpu.CompilerParams(use_tc_tiling_on_sc=False),
    scratch_types=[
        pltpu.VMEM((QS,), jnp.int32),
        pltpu.VMEM((QS, HEADS), jnp.float32),
        pltpu.SemaphoreType.DMA,
    ],
)
def _sc_gather(idx_hbm, tab_hbm, ext_hbm, idx_v, rows_v, sem):
    wid = lax.axis_index("s") * NC + lax.axis_index("c")
    base = wid * QS
    pltpu.sync_copy(idx_hbm.at[pl.ds(base, QS)], idx_v)
    pltpu.async_copy(tab_hbm.at[idx_v], rows_v, sem)
    pltpu.make_async_copy(tab_hbm.at[idx_v], rows_v, sem).wait()
    pltpu.sync_copy(rows_v, ext_hbm.at[pl.ds(base, QS)])


def _tc_body(ext_qm_ref, out_ref, ext_ref, p_ref):
    h = pl.program_id(0)
    ib = pl.program_id(1)

    @pl.when(jnp.logical_and(h == 0, ib == 0))
    def _init():
        ext_ref[...] = jnp.transpose(ext_qm_ref[...], (1, 0)) * RP_SCALE

    @pl.when(ib == 0)
    def _per_head():
        ext_b = jnp.broadcast_to(ext_ref[pl.ds(h, 1), :], (BLOCK_R, EXT2))
        p_ref[...] = pltpu.roll(ext_b, 0, axis=1, stride=1, stride_axis=0)

    q0 = pl.multiple_of(SEQ_LEN - ib * BLOCK_R, BLOCK_R)
    out_ref[0, 0] = p_ref[:, pl.ds(q0, 2 * SEQ_LEN)]


def kernel(sequence_length, table):
    del sequence_length
    bucket = jnp.asarray(_BUCKET)
    ext_qm = _sc_gather(bucket, table)
    out = pl.pallas_call(
        _tc_body,
        grid=(HEADS, SEQ_LEN // BLOCK_R),
        in_specs=[pl.BlockSpec((EXT2, HEADS), lambda h, ib: (0, 0))],
        out_specs=pl.BlockSpec(
            (1, 1, BLOCK_R, 2 * SEQ_LEN), lambda h, ib: (0, h, ib, 0)
        ),
        out_shape=jax.ShapeDtypeStruct(
            (1, HEADS, SEQ_LEN, 2 * SEQ_LEN), jnp.float32
        ),
        scratch_shapes=[
            pltpu.VMEM((HEADS, EXT2), jnp.float32),
            pltpu.VMEM((BLOCK_R, EXT2), jnp.float32),
        ],
    )(ext_qm)
    return out


# speedup vs baseline: 1.0159x; 1.0159x over previous
import functools
import math

import jax
import jax.numpy as jnp
import numpy as np
from jax import lax
from jax.experimental import pallas as pl
from jax.experimental.pallas import tpu as pltpu
from jax.experimental.pallas import tpu_sc as plsc

NUM_BUCKETS = 32
RP_MAX_DISTANCE = 128
HEADS = 16
RP_SCALE = 0.125
SEQ_LEN = 2048
EXT2 = 6400
BLOCK_R = 512
NC, NS = 2, 16
NW = NC * NS
QS = EXT2 // NW


def _bucket_table() -> np.ndarray:
    q = np.arange(EXT2, dtype=np.int64)
    n = np.maximum(SEQ_LEN - q, 0).astype(np.int64)
    max_exact = NUM_BUCKETS // 2
    nf = np.maximum(n, 1).astype(np.float64)
    val_if_large = max_exact + (
        np.log(nf / max_exact) / math.log(RP_MAX_DISTANCE / max_exact)
        * (NUM_BUCKETS - max_exact)
    ).astype(np.int64)
    val_if_large = np.minimum(val_if_large, NUM_BUCKETS - 1)
    return np.where(n < max_exact, n, val_if_large).astype(np.int32)


_BUCKET = _bucket_table()

_SC_MESH = plsc.VectorSubcoreMesh(core_axis_name="c", subcore_axis_name="s")


@functools.partial(
    pl.kernel,
    mesh=_SC_MESH,
    out_type=jax.ShapeDtypeStruct((EXT2, HEADS), jnp.float32),
    compiler_params=pltpu.CompilerParams(use_tc_tiling_on_sc=False),
    scratch_types=[
        pltpu.VMEM((QS,), jnp.int32),
        pltpu.VMEM((QS, HEADS), jnp.float32),
        pltpu.SemaphoreType.DMA,
    ],
)
def _sc_gather(idx_hbm, tab_hbm, ext_hbm, idx_v, rows_v, sem):
    wid = lax.axis_index("s") * NC + lax.axis_index("c")
    base = wid * QS
    pltpu.sync_copy(idx_hbm.at[pl.ds(base, QS)], idx_v)
    pltpu.async_copy(tab_hbm.at[idx_v], rows_v, sem)
    pltpu.make_async_copy(tab_hbm.at[idx_v], rows_v, sem).wait()
    pltpu.sync_copy(rows_v, ext_hbm.at[pl.ds(base, QS)])


def _tc_body(ext_qm_ref, out_ref, ext_ref, p_ref):
    h = pl.program_id(0)
    ib = pl.program_id(1)

    @pl.when(jnp.logical_and(h == 0, ib == 0))
    def _init():
        ext_ref[...] = jnp.transpose(ext_qm_ref[...], (1, 0)) * RP_SCALE

    @pl.when(ib == 0)
    def _per_head():
        ext_b = jnp.broadcast_to(ext_ref[pl.ds(h, 1), :], (BLOCK_R, EXT2))
        p_ref[...] = pltpu.roll(ext_b, 0, axis=1, stride=1, stride_axis=0)

    q0 = pl.multiple_of(SEQ_LEN - ib * BLOCK_R, BLOCK_R)
    out_ref[0, 0] = p_ref[:, pl.ds(q0, 2 * SEQ_LEN)]


def kernel(sequence_length, table):
    del sequence_length
    bucket = jnp.asarray(_BUCKET)
    ext_qm = _sc_gather(bucket, table)
    out = pl.pallas_call(
        _tc_body,
        grid=(HEADS, SEQ_LEN // BLOCK_R),
        in_specs=[pl.BlockSpec((EXT2, HEADS), lambda h, ib: (0, 0))],
        out_specs=pl.BlockSpec(
            (1, 1, BLOCK_R, 2 * SEQ_LEN), lambda h, ib: (0, h, ib, 0)
        ),
        out_shape=jax.ShapeDtypeStruct(
            (1, HEADS, SEQ_LEN, 2 * SEQ_LEN), jnp.float32
        ),
        scratch_shapes=[
            pltpu.VMEM((HEADS, EXT2), jnp.float32),
            pltpu.VMEM((BLOCK_R, EXT2), jnp.float32),
        ],
    )(ext_qm)
    return out
